# baseline (speedup 1.0000x reference)
import jax
import jax.numpy as jnp
from jax import lax
from jax.experimental import pallas as pl
from jax.experimental.pallas import tpu as pltpu
from jax.experimental.pallas import tpu_sc as plsc

NC = 2
NS = 16
NW = NC * NS
L = 16

NUM_W = 64
HALF = NUM_W // 2


def _make_sc_call(rows, cols, tbl_n):
    rpw = rows // NW
    nvec = cols // L

    mesh = plsc.VectorSubcoreMesh(
        core_axis_name="c", subcore_axis_name="s",
        num_cores=NC, num_subcores=NS)

    @pl.kernel(
        out_type=jax.ShapeDtypeStruct((rows, cols), jnp.float32),
        mesh=mesh,
        compiler_params=pltpu.CompilerParams(needs_layout_passes=False),
        scratch_types=[
            pltpu.VMEM((tbl_n,), jnp.float32),
            pltpu.VMEM((rpw + L,), jnp.int32),
            pltpu.VMEM((L,), jnp.float32),
            pltpu.VMEM((cols,), jnp.float32),
            pltpu.VMEM((cols,), jnp.float32),
            pltpu.VMEM((cols,), jnp.float32),
            pltpu.VMEM((cols,), jnp.float32),
            pltpu.SemaphoreType.DMA,
            pltpu.SemaphoreType.DMA,
            pltpu.SemaphoreType.DMA,
            pltpu.SemaphoreType.DMA,
        ],
    )
    def sc_fn(x_hbm, tbl_hbm, g_hbm, zrow_hbm, out_hbm,
              tbl_v, zrow_v, g_v, xb0, xb1, ob0, ob1,
              si0, si1, so0, so1):
        wid = lax.axis_index("s") * NC + lax.axis_index("c")
        base = wid * rpw
        xbuf = (xb0, xb1)
        obuf = (ob0, ob1)
        sin = (si0, si1)
        sout = (so0, so1)

        pltpu.sync_copy(g_hbm, g_v)
        pltpu.sync_copy(zrow_hbm.at[pl.ds(base, rpw)], zrow_v.at[pl.ds(0, rpw)])

        pltpu.async_copy(x_hbm.at[base], xbuf[0], sin[0])
        pltpu.async_copy(x_hbm.at[base + 1], xbuf[1], sin[1])
        pltpu.sync_copy(tbl_hbm, tbl_v)

        gv = g_v[...]
        inv_g = 1.0 / gv
        lo = -(gv * float(HALF))
        hi = gv * float(HALF - 1)
        fbias = jnp.full((L,), float(HALF), jnp.float32)

        def compute_row(j, xr, orr):
            zk = zrow_v[pl.ds(j, L)][0] - HALF
            zvec = jnp.full((L,), zk, jnp.int32)

            @pl.loop(0, nvec, unroll=4)
            def _vec(v):
                sl = pl.ds(v * L, L)
                x = xr[sl]
                xc = jnp.minimum(jnp.maximum(x, lo), hi)
                tb = xc * inv_g + fbias
                i = tb.astype(jnp.int32)
                frac = tb - i.astype(jnp.float32)
                idx0 = zvec + i
                c0 = plsc.load_gather(tbl_v, [idx0])
                c1 = plsc.load_gather(tbl_v, [idx0 + 1])
                res = c0 + frac * (c1 - c0)
                orr[sl] = jnp.where(x == xc, res, x)

        @pl.loop(0, rpw // 2)
        def _pair(j2):
            for b in range(2):
                j = j2 * 2 + b
                pltpu.make_async_copy(x_hbm.at[base + j], xbuf[b], sin[b]).wait()

                @pl.when(j2 >= 1)
                def _():
                    pltpu.make_async_copy(
                        obuf[b], out_hbm.at[base + j - 2], sout[b]).wait()

                compute_row(j, xbuf[b], obuf[b])
                pltpu.async_copy(obuf[b], out_hbm.at[base + j], sout[b])

                @pl.when(j2 < rpw // 2 - 1)
                def _():
                    pltpu.async_copy(x_hbm.at[base + j + 2], xbuf[b], sin[b])

        for b in range(2):
            pltpu.make_async_copy(
                obuf[b], out_hbm.at[base + rpw - 2 + b], sout[b]).wait()

    return sc_fn


def kernel(input, coefficients_vect, grid, zero_knot_indexes):
    b, c, d, h, w = input.shape
    rows = b * c
    cols = d * h * w
    x2 = input.reshape(rows, cols)
    zrow = jnp.tile(zero_knot_indexes.astype(jnp.int32), b)
    g16 = jnp.broadcast_to(grid.astype(jnp.float32), (L,))
    sc_fn = _make_sc_call(rows, cols, coefficients_vect.shape[0])
    out = sc_fn(x2, coefficients_vect, g16, zrow)
    return out.reshape(input.shape)


# speedup vs baseline: 1.7000x; 1.7000x over previous
import jax
import jax.numpy as jnp
from jax import lax
from jax.experimental import pallas as pl
from jax.experimental.pallas import tpu as pltpu
from jax.experimental.pallas import tpu_sc as plsc

NC = 2
NS = 16
NW = NC * NS
L = 16

NUM_W = 64
HALF = NUM_W // 2


def _make_sc_call(rows, cols, tbl_n):
    rpw = rows // NW
    nvec = cols // L

    mesh = plsc.VectorSubcoreMesh(
        core_axis_name="c", subcore_axis_name="s",
        num_cores=NC, num_subcores=NS)

    @pl.kernel(
        out_type=jax.ShapeDtypeStruct((rows, cols), jnp.float32),
        mesh=mesh,
        compiler_params=pltpu.CompilerParams(needs_layout_passes=False),
        scratch_types=[
            pltpu.VMEM((tbl_n,), jnp.float32),
            pltpu.VMEM((rpw + L,), jnp.int32),
            pltpu.VMEM((L,), jnp.float32),
            pltpu.VMEM((cols,), jnp.float32),
            pltpu.VMEM((cols,), jnp.float32),
        ],
    )
    def sc_fn(x_hbm, tbl_hbm, g_hbm, zrow_hbm, out_hbm,
              tbl_v, zrow_v, g_v, xbuf, obuf):
        wid = lax.axis_index("s") * NC + lax.axis_index("c")
        base = wid * rpw

        pltpu.sync_copy(g_hbm, g_v)
        pltpu.sync_copy(zrow_hbm.at[pl.ds(base, rpw)], zrow_v.at[pl.ds(0, rpw)])
        pltpu.sync_copy(tbl_hbm, tbl_v)

        gv = g_v[...]
        inv_g = 1.0 / gv
        lo = -(gv * float(HALF))
        hi = gv * float(HALF - 1)
        fbias = jnp.full((L,), float(HALF), jnp.float32)

        @pl.loop(0, rpw)
        def _row(j):
            pltpu.sync_copy(x_hbm.at[base + j], xbuf)
            zk = zrow_v[pl.ds(j, L)][0] - HALF
            zvec = jnp.full((L,), zk, jnp.int32)

            @plsc.parallel_loop(0, nvec, unroll=4)
            def _vec(v):
                sl = pl.ds(v * L, L)
                x = xbuf[sl]
                xc = jnp.minimum(jnp.maximum(x, lo), hi)
                tb = xc * inv_g + fbias
                i = tb.astype(jnp.int32)
                frac = tb - i.astype(jnp.float32)
                idx0 = zvec + i
                c0 = plsc.load_gather(tbl_v, [idx0])
                c1 = plsc.load_gather(tbl_v, [idx0 + 1])
                res = c0 + frac * (c1 - c0)
                obuf[sl] = jnp.where(x == xc, res, x)

            pltpu.sync_copy(obuf, out_hbm.at[base + j])

    return sc_fn


def kernel(input, coefficients_vect, grid, zero_knot_indexes):
    b, c, d, h, w = input.shape
    rows = b * c
    cols = d * h * w
    x2 = input.reshape(rows, cols)
    zrow = jnp.tile(zero_knot_indexes.astype(jnp.int32), b)
    g16 = jnp.broadcast_to(grid.astype(jnp.float32), (L,))
    sc_fn = _make_sc_call(rows, cols, coefficients_vect.shape[0])
    out = sc_fn(x2, coefficients_vect, g16, zrow)
    return out.reshape(input.shape)
